# baseline (speedup 1.0000x reference)
import functools

import jax
import jax.numpy as jnp
from jax import lax
from jax.experimental import pallas as pl
from jax.experimental.pallas import tpu as pltpu
from jax.experimental.pallas import tpu_sc as plsc

_NC = 2
_NS = 16
_NW = _NC * _NS
_LN_EPS = 1e-5


def _layer_norm_2d(x, g, b):
    m = x.mean(axis=-1, keepdims=True)
    v = ((x - m) ** 2).mean(axis=-1, keepdims=True)
    return (x - m) * lax.rsqrt(v + _LN_EPS) * g + b


def _proj_body(S, turn_t, slot_t, value_t, wt, wbs, wsv, bt, bsv, bbs,
               edge_t, etypes, turn_pre, cat_tab, edge_attr):
    f32 = jnp.float32
    bf16 = jnp.bfloat16
    ns = slot_t.shape[0]
    turn_pre[...] = jnp.dot(turn_t[...], wt[...], preferred_element_type=f32) + bt[...]
    s_sv = jnp.dot(slot_t[...], wsv[...], preferred_element_type=f32)
    s_bs = jnp.dot(slot_t[...], wbs[...], preferred_element_type=f32) * (1.0 / S)
    cat_tab[:ns, :] = jnp.concatenate([s_sv, s_bs], axis=1).astype(bf16)
    v_sv = jnp.dot(value_t[...], wsv[...], preferred_element_type=f32) + bsv[...]
    v_bs = (jnp.dot(value_t[...], wbs[...], preferred_element_type=f32)
            + bbs[...]) * (1.0 / S)
    cat_tab[ns:, :] = jnp.concatenate([v_sv, v_bs], axis=1).astype(bf16)
    ne = edge_t.shape[0]
    oh = (etypes[...] == lax.broadcasted_iota(jnp.int32, (etypes.shape[0], ne), 1)).astype(f32)
    edge_attr[...] = jnp.dot(oh, edge_t[...], preferred_element_type=f32)


def _project(S, turn_table, slot_table, value_table, W_turn, W_bs, W_sv,
             b_turn, b_sv, b_bs, edge_table, etypes_col):
    B = etypes_col.shape[0]
    H = turn_table.shape[1]
    EA = edge_table.shape[1]
    nrows = slot_table.shape[0] + value_table.shape[0]
    out_shapes = [
        jax.ShapeDtypeStruct((turn_table.shape[0], H), jnp.float32),
        jax.ShapeDtypeStruct((nrows, 2 * H), jnp.bfloat16),
        jax.ShapeDtypeStruct((B, EA), jnp.float32),
    ]
    return pl.pallas_call(
        functools.partial(_proj_body, S),
        out_shape=out_shapes,
    )(turn_table, slot_table, value_table, W_turn, W_bs, W_sv,
      b_turn, b_sv, b_bs, edge_table, etypes_col)


def _sc_gather(cat_ids, cat_i32, S, H):
    B, NI = cat_ids.shape
    b_per_w = B // _NW
    IC = 8
    HW = H // 32
    BG = 4
    BT = HW // BG

    mesh = plsc.VectorSubcoreMesh(core_axis_name="c", subcore_axis_name="s",
                                  num_cores=_NC, num_subcores=_NS)

    @functools.partial(
        pl.kernel,
        mesh=mesh,
        out_type=(
            jax.ShapeDtypeStruct((B, S, H // 2), jnp.int32),
            jax.ShapeDtypeStruct((B, 1, H), jnp.float32),
        ),
        scratch_types=[
            pltpu.VMEM((IC, NI), jnp.int32),
            pltpu.VMEM((NI, H), jnp.int32),
            pltpu.VMEM((NI, H), jnp.int32),
            pltpu.VMEM((S, H // 2), jnp.int32),
            pltpu.VMEM((8, H), jnp.float32),
            pltpu.SemaphoreType.DMA,
            pltpu.SemaphoreType.DMA,
        ],
    )
    def k(ids_hbm, cat_hbm, osv_hbm, obs_hbm,
          ids_v, bufa_v, bufb_v, osv_v, obs_v, sem0, sem1):
        wid = lax.axis_index("s") * _NC + lax.axis_index("c")
        base = wid * b_per_w
        bufs = (bufa_v, bufb_v)
        sems = (sem0, sem1)
        fzero = jnp.zeros((16,), jnp.float32)
        himask = jnp.full((16,), -65536, jnp.int32)
        half = jnp.full((16,), 0x8000, jnp.int32)

        def split(w):
            e = lax.bitcast_convert_type(lax.shift_left(w, 16), jnp.float32)
            o = lax.bitcast_convert_type(w & himask, jnp.float32)
            return e, o

        def repack(e, o):
            ei = lax.bitcast_convert_type(e, jnp.int32) + half
            oi = lax.bitcast_convert_type(o, jnp.int32) + half
            return lax.shift_right_logical(ei, 16) | (oi & himask)

        def compute(rows):
            def body_sv(s, c2):
                for h in range(HW):
                    sl = pl.ds(h * 16, 16)
                    ae, ao = split(rows[s, sl])
                    be, bo = split(rows[S + s, sl])
                    osv_v[s, sl] = repack(ae + be, ao + bo)
                return c2

            lax.fori_loop(0, S, body_sv, 0)

            for g in range(BG):
                def body_bs(s, accs):
                    new = []
                    for t in range(BT):
                        sl = pl.ds(H // 2 + (g * BT + t) * 16, 16)
                        ae, ao = split(rows[s, sl])
                        be, bo = split(rows[S + s, sl])
                        new.append(accs[2 * t] + (ae + be))
                        new.append(accs[2 * t + 1] + (ao + bo))
                    return tuple(new)

                accs = lax.fori_loop(0, S, body_bs, (fzero,) * (2 * BT))
                for t in range(BT):
                    c = g * BT + t
                    obs_v[0, pl.ds(c * 32, 16)] = accs[2 * t]
                    obs_v[0, pl.ds(c * 32 + 16, 16)] = accs[2 * t + 1]

        def body_chunk(c, carry):
            b0 = base + c * IC
            pltpu.sync_copy(ids_hbm.at[pl.ds(b0, IC)], ids_v)
            cps = [None, None]
            cps[0] = pltpu.async_copy(cat_hbm.at[ids_v.at[0]], bufs[0], sems[0])
            for j in range(IC):
                if j + 1 < IC:
                    cps[(j + 1) % 2] = pltpu.async_copy(
                        cat_hbm.at[ids_v.at[j + 1]], bufs[(j + 1) % 2],
                        sems[(j + 1) % 2])
                cps[j % 2].wait()
                compute(bufs[j % 2])
                pltpu.sync_copy(osv_v, osv_hbm.at[b0 + j])
                pltpu.sync_copy(obs_v.at[pl.ds(0, 1)], obs_hbm.at[b0 + j])
            return carry

        lax.fori_loop(0, b_per_w // IC, body_chunk, 0)

    return k(cat_ids, cat_i32)


def _ln_body(psv_ref, pbs_ref, tid_ref, tpre_ref, g_ref, b_ref, out_ref):
    bsz, S, H = psv_ref.shape
    nt = tpre_ref.shape[0]
    f32 = jnp.float32
    g = g_ref[...]
    b = b_ref[...]
    x = psv_ref[...].astype(f32).reshape(bsz * S, H)
    out_ref[:, 2:, :] = _layer_norm_2d(x, g, b).reshape(bsz, S, H)
    bs = pbs_ref[...].reshape(bsz, H // 32, 2, 16)
    bs = jnp.stack([bs[:, :, 0, :], bs[:, :, 1, :]], axis=3).reshape(bsz, H)
    out_ref[:, 1, :] = _layer_norm_2d(bs, g, b)
    oh = (tid_ref[...] == lax.broadcasted_iota(jnp.int32, (bsz, nt), 1)).astype(f32)
    t = jnp.dot(oh, tpre_ref[...], preferred_element_type=f32)
    out_ref[:, 0, :] = _layer_norm_2d(t, g, b)


def _ln_phase(P_sv, P_bs, tids_col, turn_pre, g_row, b_row):
    B, S, H = P_sv.shape
    bsz = 64
    grid = B // bsz
    return pl.pallas_call(
        _ln_body,
        grid=(grid,),
        in_specs=[
            pl.BlockSpec((bsz, S, H), lambda i: (i, 0, 0)),
            pl.BlockSpec((bsz, H), lambda i: (i, 0)),
            pl.BlockSpec((bsz, 1), lambda i: (i, 0)),
            pl.BlockSpec(turn_pre.shape, lambda i: (0, 0)),
            pl.BlockSpec((1, H), lambda i: (0, 0)),
            pl.BlockSpec((1, H), lambda i: (0, 0)),
        ],
        out_specs=pl.BlockSpec((bsz, S + 2, H), lambda i: (i, 0, 0)),
        out_shape=jax.ShapeDtypeStruct((B, S + 2, H), jnp.float32),
    )(P_sv, P_bs, tids_col, turn_pre, g_row, b_row)


def kernel(turn_ids, slot_ids, value_ids, edge_types, turn_table, slot_table,
           value_table, edge_table, W_turn, b_turn, W_bs, b_bs, W_sv, b_sv,
           ln_g, ln_b):
    B = turn_ids.shape[0]
    S = slot_ids.shape[1]
    H = turn_table.shape[1]

    i32 = jnp.int32
    tids_col = jnp.asarray(turn_ids, i32).reshape(B, 1)
    sids = jnp.asarray(slot_ids, i32)
    vids = jnp.asarray(value_ids, i32)
    etypes_col = jnp.asarray(edge_types, i32).reshape(B, 1)

    turn_pre, cat_tab, edge_attr = _project(
        S, turn_table, slot_table, value_table, W_turn, W_bs, W_sv,
        b_turn.reshape(1, H), b_sv.reshape(1, H), b_bs.reshape(1, H),
        edge_table, etypes_col)

    nt = cat_tab.shape[0]
    cat_i32 = lax.bitcast_convert_type(cat_tab.reshape(nt, H, 2), i32)

    ni = (2 * S + 7) // 8 * 8
    pad = jnp.zeros((B, ni - 2 * S), i32)
    cat_ids = jnp.concatenate([sids, vids + slot_table.shape[0], pad], axis=1)

    P_sv, P_bs = _sc_gather(cat_ids, cat_i32, S, H)

    P_sv_bf = lax.bitcast_convert_type(P_sv, jnp.bfloat16).reshape(B, S, H)
    node_features = _ln_phase(
        P_sv_bf, P_bs.reshape(B, H), tids_col, turn_pre,
        ln_g.reshape(1, H), ln_b.reshape(1, H))
    return node_features, edge_attr
